# baseline (speedup 1.0000x reference)
import functools

import jax
import jax.numpy as jnp
from jax import lax
from jax.experimental import pallas as pl
from jax.experimental.pallas import tpu as pltpu
from jax.experimental.pallas import tpu_sc as plsc

_PIX = 32768

_NW = 32
_CHUNK = 128
_NBUF = 8


def _lse_block(pred_ref, msk_ref, sum_ref, cnt_ref):
    b = pl.program_id(0)
    j = pl.program_id(1)

    @pl.when(jnp.logical_and(b == 0, j == 0))
    def _init():
        sum_ref[0, 0] = jnp.float32(0.0)
        cnt_ref[0, 0] = jnp.float32(0.0)

    x = pred_ref[:, :]
    m = msk_ref[0, 0, :]

    mx = jnp.max(x, axis=0)
    s = jnp.sum(jnp.exp(x - mx[None, :]), axis=0)
    lse = mx + jnp.log(s)

    sum_ref[0, 0] += jnp.sum(m * lse)
    cnt_ref[0, 0] += jnp.sum(m)


def _tc_lse(pred2, msk3, B, C, nb):
    return pl.pallas_call(
        _lse_block,
        grid=(B, nb),
        in_specs=[
            pl.BlockSpec((C, _PIX), lambda b, j: (b, j)),
            pl.BlockSpec((1, 1, _PIX), lambda b, j, nb=nb: (b * nb + j, 0, 0)),
        ],
        out_specs=[
            pl.BlockSpec(memory_space=pltpu.SMEM),
            pl.BlockSpec(memory_space=pltpu.SMEM),
        ],
        out_shape=[
            jax.ShapeDtypeStruct((1, 1), jnp.float32),
            jax.ShapeDtypeStruct((1, 1), jnp.float32),
        ],
    )(pred2, msk3)


def _sc_gather_sum(pred_flat, tgt_flat, msk_flat, C, HW):
    npix = tgt_flat.shape[0]
    per_w = npix // _NW
    n_outer = per_w // (_CHUNK * _NBUF)
    chw = C * HW

    mesh = plsc.VectorSubcoreMesh(core_axis_name="c", subcore_axis_name="s")

    @functools.partial(
        pl.kernel,
        mesh=mesh,
        out_type=jax.ShapeDtypeStruct((_NW, 16), jnp.float32),
        scratch_types=[
            pltpu.VMEM((per_w,), jnp.int32),
            pltpu.VMEM((per_w,), jnp.float32),
            pltpu.VMEM((_NBUF, _CHUNK), jnp.int32),
            pltpu.VMEM((_NBUF, _CHUNK), jnp.float32),
            pltpu.VMEM((16,), jnp.float32),
            pltpu.SemaphoreType.DMA,
            pltpu.SemaphoreType.DMA,
        ],
    )
    def sc_fn(pred_hbm, tgt_hbm, msk_hbm, out_hbm,
              tgt_v, msk_v, idx_v, rows_v, acc_v, sem_in, sem_g):
        wid = lax.axis_index("s") * 2 + lax.axis_index("c")
        base = wid * per_w
        bidx = base // HW
        base_flat = bidx * chw + (base % HW)

        cp1 = pltpu.async_copy(tgt_hbm.at[pl.ds(base, per_w)], tgt_v, sem_in)
        cp2 = pltpu.async_copy(msk_hbm.at[pl.ds(base, per_w)], msk_v, sem_in)
        cp1.wait()
        cp2.wait()
        acc_v[...] = jnp.zeros((16,), jnp.float32)
        lane = lax.iota(jnp.int32, 16)

        def outer(o, _):
            obase = o * (_CHUNK * _NBUF)
            for j in range(_NBUF):
                coff = obase + j * _CHUNK
                for k in range(_CHUNK // 16):
                    t16 = tgt_v[pl.ds(coff + k * 16, 16)]
                    idx_v[j, pl.ds(k * 16, 16)] = (
                        t16 * HW + (base_flat + coff + k * 16) + lane)
            gathers = []
            for j in range(_NBUF):
                gathers.append(
                    pltpu.async_copy(pred_hbm.at[idx_v.at[j]], rows_v.at[j],
                                     sem_g))
            for j in range(_NBUF):
                gathers[j].wait()
            for j in range(_NBUF):
                coff = obase + j * _CHUNK
                for k in range(_CHUNK // 16):
                    r16 = rows_v[j, pl.ds(k * 16, 16)]
                    m16 = msk_v[pl.ds(coff + k * 16, 16)]
                    acc_v[...] += r16 * m16
            return ()

        lax.fori_loop(0, n_outer, outer, (), unroll=False)
        pltpu.sync_copy(acc_v, out_hbm.at[wid])

    return sc_fn(pred_flat, tgt_flat, msk_flat)


def kernel(pred, target, label_mask):
    B, C, H, W = pred.shape
    HW = H * W
    nb = HW // _PIX

    pred2 = pred.reshape(B * C, HW)
    mskf = label_mask.astype(jnp.float32)

    partials = _sc_gather_sum(pred.reshape(-1), target.astype(jnp.int32).reshape(-1),
                              mskf.reshape(-1), C, HW)

    total_lse, count = _tc_lse(pred2, mskf.reshape(B * nb, 1, _PIX), B, C, nb)

    total = total_lse[0, 0] - jnp.sum(partials)
    count = count[0, 0]
    safe = jnp.where(count > 0, count, jnp.float32(1.0))
    return jnp.where(count > 0, total / safe, jnp.float32(0.0))


# speedup vs baseline: 1.0083x; 1.0083x over previous
import functools

import jax
import jax.numpy as jnp
from jax import lax
from jax.experimental import pallas as pl
from jax.experimental.pallas import tpu as pltpu
from jax.experimental.pallas import tpu_sc as plsc

_PIX = 32768

_NW = 32
_CHUNK = 128
_NBUF = 8


def _lse_block(pred_ref, msk_ref, sum_ref, cnt_ref):
    b = pl.program_id(0)
    j = pl.program_id(1)

    @pl.when(jnp.logical_and(b == 0, j == 0))
    def _init():
        sum_ref[0, 0] = jnp.float32(0.0)
        cnt_ref[0, 0] = jnp.float32(0.0)

    x = pred_ref[:, :]
    m = msk_ref[0, 0, :]

    mx = jnp.max(x, axis=0)
    s = jnp.sum(jnp.exp(x - mx[None, :]), axis=0)
    lse = mx + jnp.log(s)

    sum_ref[0, 0] += jnp.sum(m * lse)
    cnt_ref[0, 0] += jnp.sum(m)


def _tc_lse(pred2, msk3, B, C, nb):
    return pl.pallas_call(
        _lse_block,
        grid=(B, nb),
        in_specs=[
            pl.BlockSpec((C, _PIX), lambda b, j: (b, j)),
            pl.BlockSpec((1, 1, _PIX), lambda b, j, nb=nb: (b * nb + j, 0, 0)),
        ],
        out_specs=[
            pl.BlockSpec(memory_space=pltpu.SMEM),
            pl.BlockSpec(memory_space=pltpu.SMEM),
        ],
        out_shape=[
            jax.ShapeDtypeStruct((1, 1), jnp.float32),
            jax.ShapeDtypeStruct((1, 1), jnp.float32),
        ],
    )(pred2, msk3)


def _sc_gather_sum(pred_flat, tgt, mskf, C, H, W):
    B = tgt.shape[0]
    HW = H * W
    npix = B * HW
    per_w = npix // _NW
    rows_w = per_w // W
    n_outer = per_w // (_CHUNK * _NBUF)
    cpr = W // _CHUNK
    chw = C * HW

    mesh = plsc.VectorSubcoreMesh(core_axis_name="c", subcore_axis_name="s")

    @functools.partial(
        pl.kernel,
        mesh=mesh,
        out_type=jax.ShapeDtypeStruct((_NW, 16), jnp.float32),
        scratch_types=[
            pltpu.VMEM((rows_w, W), jnp.int32),
            pltpu.VMEM((rows_w, W), jnp.float32),
            pltpu.VMEM((_NBUF, _CHUNK), jnp.int32),
            pltpu.VMEM((_NBUF, _CHUNK), jnp.float32),
            pltpu.VMEM((16,), jnp.float32),
            pltpu.SemaphoreType.DMA,
            pltpu.SemaphoreType.DMA,
        ],
    )
    def sc_fn(pred_hbm, tgt_hbm, msk_hbm, out_hbm,
              tgt_v, msk_v, idx_v, rows_v, acc_v, sem_in, sem_g):
        wid = lax.axis_index("s") * 2 + lax.axis_index("c")
        base = wid * per_w
        bidx = base // HW
        base_flat = bidx * chw + (base % HW)
        r0 = pl.multiple_of((base % HW) // W, rows_w)

        cp1 = pltpu.async_copy(tgt_hbm.at[bidx, pl.ds(r0, rows_w)], tgt_v,
                               sem_in)
        cp2 = pltpu.async_copy(msk_hbm.at[bidx, pl.ds(r0, rows_w)], msk_v,
                               sem_in)
        cp1.wait()
        cp2.wait()
        acc_v[...] = jnp.zeros((16,), jnp.float32)
        lane = lax.iota(jnp.int32, 16)

        def outer(o, _):
            obase = o * (_CHUNK * _NBUF)
            row0 = obase // W
            for j in range(_NBUF):
                coff = obase + j * _CHUNK
                row = row0 + j // cpr
                col = (j % cpr) * _CHUNK
                for k in range(_CHUNK // 16):
                    t16 = tgt_v[row, pl.ds(col + k * 16, 16)]
                    idx_v[j, pl.ds(k * 16, 16)] = (
                        t16 * HW + (base_flat + coff + k * 16) + lane)
            gathers = []
            for j in range(_NBUF):
                gathers.append(
                    pltpu.async_copy(pred_hbm.at[idx_v.at[j]], rows_v.at[j],
                                     sem_g))
            for j in range(_NBUF):
                gathers[j].wait()
            for j in range(_NBUF):
                row = row0 + j // cpr
                col = (j % cpr) * _CHUNK
                for k in range(_CHUNK // 16):
                    r16 = rows_v[j, pl.ds(k * 16, 16)]
                    m16 = msk_v[row, pl.ds(col + k * 16, 16)]
                    acc_v[...] += r16 * m16
            return ()

        lax.fori_loop(0, n_outer, outer, (), unroll=False)
        pltpu.sync_copy(acc_v, out_hbm.at[wid])

    return sc_fn(pred_flat, tgt, mskf)


def kernel(pred, target, label_mask):
    B, C, H, W = pred.shape
    HW = H * W
    nb = HW // _PIX

    pred2 = pred.reshape(B * C, HW)
    mskf = label_mask.astype(jnp.float32)

    partials = _sc_gather_sum(pred.reshape(-1), target.astype(jnp.int32),
                              mskf, C, H, W)

    total_lse, count = _tc_lse(pred2, mskf.reshape(B * nb, 1, _PIX), B, C, nb)

    total = total_lse[0, 0] - jnp.sum(partials)
    count = count[0, 0]
    safe = jnp.where(count > 0, count, jnp.float32(1.0))
    return jnp.where(count > 0, total / safe, jnp.float32(0.0))
